# Initial kernel scaffold; baseline (speedup 1.0000x reference)
import jax
import jax.numpy as jnp
from jax.experimental import pallas as pl


def kernel(x, scalars, edge_index, batch, W1, b1, W2, b2):
    raise NotImplementedError("write your pallas kernel here")


# speedup vs baseline: 5.9902x; 5.9902x over previous
import functools

import jax
import jax.numpy as jnp
from jax import lax
from jax.experimental import pallas as pl
from jax.experimental.pallas import tpu as pltpu
from jax.experimental.pallas import tpu_sc as plsc

N = 10000
E = 320000
DF = 128
NS = 8
NG = 16
DH = 128

NB = 10
BN = N // NB

NC = 2
NSUB = 16
NW = NC * NSUB
EPW = E // NW
K = 80
NCH = EPW // K
NPAD = 10240
RPT = NPAD // NSUB


def _pre_body(x_ref, bt_ref, sc_ref, W1_ref, b1_ref, A_ref, B_ref):
    xb = x_ref[...]
    bt = bt_ref[0, 0, :]
    oh = (bt[:, None] == lax.broadcasted_iota(jnp.int32, (1, NG), 1)
          ).astype(jnp.float32)
    s = jnp.dot(oh, sc_ref[...], preferred_element_type=jnp.float32)
    w1ax = W1_ref[0:DF, :]
    w1as = W1_ref[DF:DF + NS, :]
    w1bx = W1_ref[DF + NS:2 * DF + NS, :]
    w1bs = W1_ref[2 * DF + NS:, :]
    b1 = b1_ref[0, :]
    A_ref[...] = (jnp.dot(xb, w1ax - w1bx, preferred_element_type=jnp.float32)
                  + jnp.dot(s, w1as - w1bs, preferred_element_type=jnp.float32)
                  + b1[None, :])
    B_ref[...] = (jnp.dot(xb, w1bx, preferred_element_type=jnp.float32)
                  + jnp.dot(s, w1bs, preferred_element_type=jnp.float32))


_pre = pl.pallas_call(
    _pre_body,
    grid=(NB,),
    in_specs=[
        pl.BlockSpec((BN, DF), lambda i: (i, 0)),
        pl.BlockSpec((1, 1, BN), lambda i: (i, 0, 0)),
        pl.BlockSpec((NG, NS), lambda i: (0, 0)),
        pl.BlockSpec((2 * (DF + NS), DH), lambda i: (0, 0)),
        pl.BlockSpec((1, DH), lambda i: (0, 0)),
    ],
    out_specs=[pl.BlockSpec((BN, DH), lambda i: (i, 0)),
               pl.BlockSpec((BN, DH), lambda i: (i, 0))],
    out_shape=[jax.ShapeDtypeStruct((N, DH), jnp.float32),
               jax.ShapeDtypeStruct((N, DH), jnp.float32)],
)


_mesh = plsc.VectorSubcoreMesh(core_axis_name="c", subcore_axis_name="s")


@functools.partial(
    pl.kernel,
    out_type=(jax.ShapeDtypeStruct((NC, NPAD, DH), jnp.float32),
              jax.ShapeDtypeStruct((NC, NPAD, 16), jnp.float32)),
    mesh=_mesh,
    scratch_types=[
        pltpu.VMEM((K,), jnp.int32),
        pltpu.VMEM((K,), jnp.int32),
        pltpu.VMEM((K, DH), jnp.float32),
        pltpu.VMEM((K, DH), jnp.float32),
        pltpu.VMEM((K, 16), jnp.float32),
        pltpu.VMEM((K, DH), jnp.float32),
        pltpu.VMEM_SHARED((NPAD, DH), jnp.float32),
        pltpu.VMEM_SHARED((NPAD, 16), jnp.float32),
        pltpu.SemaphoreType.DMA,
        pltpu.SemaphoreType.DMA,
    ],
    compiler_params=pltpu.CompilerParams(use_tc_tiling_on_sc=False),
)
def _edge(A_hbm, B_hbm, src_hbm, dst_hbm, S_out, Dg_out,
          dsti_c, srci_c, arows, brows, ones_v, zbuf, S_sh, D_sh, sem1, sem2):
    cid = lax.axis_index("c")
    sid = lax.axis_index("s")
    wid = sid * NC + cid

    zv = jnp.zeros((16,), jnp.float32)
    ov = jnp.ones((16,), jnp.float32)

    def zrow(i, _):
        for t in range(DH // 16):
            zbuf[i, pl.ds(t * 16, 16)] = zv
        return 0
    lax.fori_loop(0, K, zrow, 0)

    def zrow16(i, _):
        ones_v[i, :] = zv
        return 0
    lax.fori_loop(0, K, zrow16, 0)

    def zs(t, _):
        base = sid * RPT + t * K
        pltpu.sync_copy(zbuf, S_sh.at[pl.ds(base, K)])
        pltpu.sync_copy(ones_v, D_sh.at[pl.ds(base, K)])
        return 0
    lax.fori_loop(0, RPT // K, zs, 0)
    plsc.subcore_barrier()

    def orow(i, _):
        ones_v[i, :] = ov
        return 0
    lax.fori_loop(0, K, orow, 0)

    def chunk(j, _):
        pltpu.sync_copy(dst_hbm.at[wid, j], dsti_c)
        pltpu.sync_copy(src_hbm.at[wid, j], srci_c)
        c1 = pltpu.async_copy(A_hbm.at[dsti_c], arows, sem1)
        c2 = pltpu.async_copy(B_hbm.at[srci_c], brows, sem2)
        c1.wait()
        c2.wait()

        def row(i, _):
            for t in range(DH // 16):
                sl = pl.ds(t * 16, 16)
                arows[i, sl] = jnp.maximum(arows[i, sl] + brows[i, sl], 0.0)
            return 0
        lax.fori_loop(0, K, row, 0)

        pltpu.sync_copy(arows, S_sh.at[dsti_c], add=True)
        pltpu.sync_copy(ones_v, D_sh.at[dsti_c], add=True)
        return 0
    lax.fori_loop(0, NCH, chunk, 0)
    plsc.subcore_barrier()

    def wb(t, _):
        base = sid * RPT + t * K
        pltpu.sync_copy(S_sh.at[pl.ds(base, K)], zbuf)
        pltpu.sync_copy(zbuf, S_out.at[cid, pl.ds(base, K)])
        pltpu.sync_copy(D_sh.at[pl.ds(base, K)], ones_v)
        pltpu.sync_copy(ones_v, Dg_out.at[cid, pl.ds(base, K)])
        return 0
    lax.fori_loop(0, RPT // K, wb, 0)


def _fin_body(S_ref, Dg_ref, W2_ref, b2_ref, o_ref):
    s = S_ref[0] + S_ref[1]
    deg = Dg_ref[0, :, 0] + Dg_ref[1, :, 0]
    o_ref[...] = (jnp.dot(s, W2_ref[...], preferred_element_type=jnp.float32)
                  + deg[:, None] * b2_ref[0, :][None, :])


_fin = pl.pallas_call(
    _fin_body,
    grid=(NB,),
    in_specs=[
        pl.BlockSpec((NC, BN, DH), lambda i: (0, i, 0)),
        pl.BlockSpec((NC, BN, 16), lambda i: (0, i, 0)),
        pl.BlockSpec((DH, DH), lambda i: (0, 0)),
        pl.BlockSpec((1, DH), lambda i: (0, 0)),
    ],
    out_specs=pl.BlockSpec((BN, DH), lambda i: (i, 0)),
    out_shape=jax.ShapeDtypeStruct((N, DH), jnp.float32),
)


def kernel(x, scalars, edge_index, batch, W1, b1, W2, b2):
    srcr = edge_index[0].reshape(NW, NCH, K)
    dstr = edge_index[1].reshape(NW, NCH, K)
    bt3 = batch.reshape(NB, 1, BN)
    A, B = _pre(x, bt3, scalars, W1, b1.reshape(1, DH))
    S, Dg = _edge(A, B, srcr, dstr)
    return _fin(S, Dg, W2, b2.reshape(1, DH))
